# baseline (speedup 1.0000x reference)
import functools

import jax
import jax.numpy as jnp
from jax.experimental import pallas as pl
from jax.experimental.pallas import tpu as pltpu
from jax.experimental.pallas import tpu_sc as plsc

EPS = 1e-5
K_CONST = 0.1
INNER_RADIUS = 2.0 * K_CONST / (1.0 + (1.0 + 4.0 * K_CONST * K_CONST) ** 0.5)
MAX_NORM = 1.0 - EPS

_GATHER_WINDOW = 128


def _sc_gather(table, flat_idx):
    n_idx = flat_idx.shape[1]
    dim = table.shape[1]
    mesh = plsc.VectorSubcoreMesh(core_axis_name="core",
                                  subcore_axis_name="subcore")

    @functools.partial(
        pl.kernel,
        out_type=jax.ShapeDtypeStruct((n_idx, dim), table.dtype),
        mesh=mesh,
        compiler_params=pltpu.CompilerParams(use_tc_tiling_on_sc=False),
    )
    def gather_kernel(table_hbm, idx_hbm, out_hbm):
        def body(idx_vmem, out_vmem):
            pltpu.sync_copy(table_hbm.at[idx_vmem.at[0]], out_vmem)

        pltpu.emit_pipeline(
            body,
            grid=(n_idx // _GATHER_WINDOW,),
            in_specs=[pl.BlockSpec((1, _GATHER_WINDOW),
                                   index_map=lambda i: (0, i))],
            out_specs=[pl.BlockSpec((_GATHER_WINDOW, dim),
                                    index_map=lambda i: (i, 0))],
            core_axis_name=("core", "subcore"),
            dimension_semantics=(pltpu.PARALLEL,),
        )(idx_hbm, out_hbm)

    return gather_kernel(table, flat_idx)


_PI = 3.14159265358979323846
_HALF_PI = 1.57079632679489661923


def _acos(x):
    ax = jnp.abs(x)
    p = -0.0012624911
    p = p * ax + 0.0066700901
    p = p * ax - 0.0170881256
    p = p * ax + 0.0308918810
    p = p * ax - 0.0501743046
    p = p * ax + 0.0889789874
    p = p * ax - 0.2145988016
    p = p * ax + 1.5707963050
    pos = jnp.sqrt(jnp.maximum(1.0 - ax, 0.0)) * p
    return jnp.where(x >= 0.0, pos, _PI - pos)


def _asin(x):
    return _HALF_PI - _acos(x)


def _energy_body(e_ref, out_ref):
    e = e_ref[...]
    sq = jnp.sum(e * e, axis=-1, keepdims=True)
    norm = jnp.sqrt(sq)
    scale = jnp.minimum(1.0, MAX_NORM / jnp.maximum(norm, 1e-12))
    en = e * scale
    o = en[:, 1:, :]
    s = en[:, 0:1, :]
    sq_o = jnp.sum(o * o, axis=-1)
    sq_s = jnp.sum(s * s, axis=-1)
    dot = jnp.sum(o * s, axis=-1)
    edist = jnp.sqrt(jnp.sum((o - s) * (o - s), axis=-1))
    norm_s = jnp.sqrt(sq_s)
    num = dot * (1.0 + sq_s) - sq_s * (1.0 + sq_o)
    denom = norm_s * edist * jnp.sqrt(
        jnp.maximum(1.0 + sq_s * sq_o - 2.0 * dot, 1e-12))
    cos_angle = num / jnp.maximum(denom, EPS)
    angle = _acos(jnp.clip(cos_angle, -1.0 + EPS, 1.0 - EPS))
    arg = INNER_RADIUS * (1.0 - sq_o) / jnp.sqrt(jnp.maximum(sq_o, 1e-12))
    half_ap = _asin(jnp.clip(arg, -1.0 + EPS, 1.0 - EPS))
    out_ref[...] = jnp.maximum(angle - half_ap, 0.0)


def _tc_energy(e, b_blk):
    batch, nsamp, dim = e.shape
    grid = (batch // b_blk,)
    return pl.pallas_call(
        _energy_body,
        grid=grid,
        in_specs=[pl.BlockSpec((b_blk, nsamp, dim), lambda i: (i, 0, 0))],
        out_specs=pl.BlockSpec((b_blk, nsamp - 1), lambda i: (i, 0)),
        out_shape=jax.ShapeDtypeStruct((batch, nsamp - 1), e.dtype),
    )(e)


def kernel(inputs, table):
    batch, nsamp = inputs.shape
    dim = table.shape[1]
    flat_idx = inputs.reshape(1, batch * nsamp)
    gathered = _sc_gather(table, flat_idx)
    e = gathered.reshape(batch, nsamp, dim)
    return _tc_energy(e, b_blk=128)


# speedup vs baseline: 1.7963x; 1.7963x over previous
import functools

import jax
import jax.numpy as jnp
from jax.experimental import pallas as pl
from jax.experimental.pallas import tpu as pltpu
from jax.experimental.pallas import tpu_sc as plsc

EPS = 1e-5
K_CONST = 0.1
INNER_RADIUS = 2.0 * K_CONST / (1.0 + (1.0 + 4.0 * K_CONST * K_CONST) ** 0.5)
MAX_NORM = 1.0 - EPS

_GATHER_WINDOW = 128


def _sc_gather(table, flat_idx):
    n_idx = flat_idx.shape[1]
    dim = table.shape[1]
    mesh = plsc.VectorSubcoreMesh(core_axis_name="core",
                                  subcore_axis_name="subcore")

    @functools.partial(
        pl.kernel,
        out_type=jax.ShapeDtypeStruct((n_idx, dim), table.dtype),
        mesh=mesh,
        compiler_params=pltpu.CompilerParams(use_tc_tiling_on_sc=False),
    )
    def gather_kernel(table_hbm, idx_hbm, out_hbm):
        def body(idx_vmem, out_vmem):
            pltpu.sync_copy(table_hbm.at[idx_vmem.at[0]], out_vmem)

        pltpu.emit_pipeline(
            body,
            grid=(n_idx // _GATHER_WINDOW,),
            in_specs=[pl.BlockSpec((1, _GATHER_WINDOW),
                                   index_map=lambda i: (0, i))],
            out_specs=[pl.BlockSpec((_GATHER_WINDOW, dim),
                                    index_map=lambda i: (i, 0))],
            core_axis_name=("core", "subcore"),
            dimension_semantics=(pltpu.PARALLEL,),
        )(idx_hbm, out_hbm)

    return gather_kernel(table, flat_idx)


_DET_COLS = 15872


def _detile_body(t_ref, out_ref):
    x = t_ref[...]
    dim, cols = x.shape
    pack = 128 // dim
    rows = cols * dim // 128
    parts = x.T.reshape(rows, pack, dim)
    out_ref[...] = jnp.concatenate(
        [parts[:, u, :] for u in range(pack)], axis=-1)


def _tc_detile(table_t):
    d, v = table_t.shape
    n_k = (v + _DET_COLS - 1) // _DET_COLS
    rows = _DET_COLS * d // 128
    return pl.pallas_call(
        _detile_body,
        grid=(n_k,),
        in_specs=[pl.BlockSpec((d, _DET_COLS), lambda i: (0, i))],
        out_specs=pl.BlockSpec((rows, 128), lambda i: (i, 0)),
        out_shape=jax.ShapeDtypeStruct((v * d // 128, 128), table_t.dtype),
        compiler_params=pltpu.CompilerParams(
            dimension_semantics=("parallel",)),
    )(table_t)


_PI = 3.14159265358979323846
_HALF_PI = 1.57079632679489661923


def _acos(x):
    ax = jnp.abs(x)
    p = -0.0012624911
    p = p * ax + 0.0066700901
    p = p * ax - 0.0170881256
    p = p * ax + 0.0308918810
    p = p * ax - 0.0501743046
    p = p * ax + 0.0889789874
    p = p * ax - 0.2145988016
    p = p * ax + 1.5707963050
    pos = jnp.sqrt(jnp.maximum(1.0 - ax, 0.0)) * p
    return jnp.where(x >= 0.0, pos, _PI - pos)


def _asin(x):
    return _HALF_PI - _acos(x)


def _energy_body(e_ref, out_ref):
    e = e_ref[...]
    e2 = jnp.swapaxes(e, 1, 2)
    sq_raw = jnp.sum(e2 * e2, axis=1)
    norm = jnp.sqrt(sq_raw)
    scale = jnp.minimum(1.0, MAX_NORM / jnp.maximum(norm, 1e-12))
    dot_raw = jnp.sum(e2[:, :, 1:] * e2[:, :, 0:1], axis=1)
    so = scale[:, 1:]
    ss = scale[:, 0:1]
    sq_o = sq_raw[:, 1:] * so * so
    sq_s = sq_raw[:, 0:1] * ss * ss
    dot = dot_raw * so * ss
    edist = jnp.sqrt(jnp.maximum(sq_o + sq_s - 2.0 * dot, 0.0))
    norm_s = jnp.sqrt(sq_s)
    num = dot * (1.0 + sq_s) - sq_s * (1.0 + sq_o)
    denom = norm_s * edist * jnp.sqrt(
        jnp.maximum(1.0 + sq_s * sq_o - 2.0 * dot, 1e-12))
    cos_angle = num / jnp.maximum(denom, EPS)
    angle = _acos(jnp.clip(cos_angle, -1.0 + EPS, 1.0 - EPS))
    arg = INNER_RADIUS * (1.0 - sq_o) / jnp.sqrt(jnp.maximum(sq_o, 1e-12))
    half_ap = _asin(jnp.clip(arg, -1.0 + EPS, 1.0 - EPS))
    out_ref[...] = jnp.maximum(angle - half_ap, 0.0)


def _tc_energy(e, b_blk):
    batch, nsamp, dim = e.shape
    grid = (batch // b_blk,)
    return pl.pallas_call(
        _energy_body,
        grid=grid,
        in_specs=[pl.BlockSpec((b_blk, nsamp, dim), lambda i: (i, 0, 0))],
        out_specs=pl.BlockSpec((b_blk, nsamp - 1), lambda i: (i, 0)),
        out_shape=jax.ShapeDtypeStruct((batch, nsamp - 1), e.dtype),
        compiler_params=pltpu.CompilerParams(
            dimension_semantics=("parallel",)),
    )(e)


def kernel(inputs, table):
    batch, nsamp = inputs.shape
    nobj, dim = table.shape
    flat_idx = inputs.reshape(1, batch * nsamp)
    table_lin = _tc_detile(table.T).reshape(nobj, dim)
    gathered = _sc_gather(table_lin, flat_idx)
    e = gathered.reshape(batch, nsamp, dim)
    return _tc_energy(e, b_blk=256)
